# baseline (speedup 1.0000x reference)
import functools

import jax
import jax.numpy as jnp
from jax.experimental import pallas as pl
from jax.experimental.pallas import tpu as pltpu

_VALID_BIAS = -100.0
_DN_NT = (((1,), (1,)), ((), ()))
_DN_TN = (((0,), (0,)), ((), ()))


def _body(x_ref, vis_ref, mask_ref, wu_ref, wk_ref, wv_ref, bu_ref,
          out_ref, k_ref, v_ref, s_ref, p_ref, w_ref, xs_ref,
          *, nvis, tile_a, na, tile_b):
    i = pl.program_id(0)
    vp, t = s_ref.shape
    dim = x_ref.shape[1]
    scale = jax.lax.rsqrt(jnp.float32(dim))

    @pl.when(i == 0)
    def _setup():
        vis = jnp.concatenate(
            [vis_ref[...], jnp.zeros((vp - nvis, vis_ref.shape[1]),
                                     jnp.float32)], axis=0)
        up = jax.lax.dot_general(vis, wu_ref[...], _DN_NT,
                                 preferred_element_type=jnp.float32)
        up = up + bu_ref[...]
        k_ref[...] = jax.lax.dot_general(up, wk_ref[...], _DN_NT,
                                         preferred_element_type=jnp.float32)
        v_ref[...] = jax.lax.dot_general(up, wv_ref[...], _DN_NT,
                                         preferred_element_type=jnp.float32
                                         ).astype(jnp.bfloat16)
        mask = jnp.concatenate(
            [mask_ref[...].astype(jnp.float32),
             jnp.zeros((vp - nvis, t), jnp.float32)], axis=0)
        cnt = jnp.sum(mask, axis=0, keepdims=True)
        npad = jnp.max(cnt) - cnt
        rowid = jax.lax.broadcasted_iota(jnp.int32, (vp, t), 0)
        w_ref[...] = jnp.where(rowid == nvis, npad, mask)

    @pl.when(i < na)
    def _scores():
        xt = x_ref[...]
        xs_ref[pl.ds(i * tile_a, tile_a), :] = xt
        s_ref[:, pl.ds(i * tile_a, tile_a)] = jax.lax.dot_general(
            k_ref[...], xt, _DN_NT, preferred_element_type=jnp.float32)

    @pl.when(i == na)
    def _normalize():
        w = w_ref[...]
        rowid = jax.lax.broadcasted_iota(jnp.int32, (vp, t), 0)
        bias = jnp.where(rowid < nvis, jnp.float32(_VALID_BIAS),
                         jnp.float32(0.0))
        lm = jnp.where(w > 0, s_ref[...] * scale + bias, -jnp.inf)
        e = w * jnp.exp(lm - jnp.max(lm))
        p_ref[...] = (e / jnp.sum(e)).astype(jnp.bfloat16)

    @pl.when(i >= na)
    def _combine():
        j = i - na
        p = p_ref[:, pl.ds(j * tile_b, tile_b)]
        attn = jax.lax.dot_general(p, v_ref[...], _DN_TN,
                                   preferred_element_type=jnp.float32)
        out_ref[...] = attn + xs_ref[pl.ds(j * tile_b, tile_b), :]


def kernel(x, vision, attention_mask, Wk, Wv, Wu, bu):
    b, t, dim = x.shape
    v = vision.shape[1]
    cv = vision.shape[2]
    vp = ((v + 1 + 7) // 8) * 8
    tile_a = 1024
    tile_b = 512
    na = t // tile_a
    nb = t // tile_b

    full = lambda shape: pl.BlockSpec(shape, lambda i: (0, 0))
    out = pl.pallas_call(
        functools.partial(_body, nvis=v, tile_a=tile_a, na=na, tile_b=tile_b),
        grid=(na + nb,),
        in_specs=[
            pl.BlockSpec((tile_a, dim),
                         lambda i: (jnp.where(i < na, i, na - 1), 0)),
            full((v, cv)),
            full((v, t)),
            full((dim, cv)),
            full((dim, dim)),
            full((dim, dim)),
            full((1, dim)),
        ],
        out_specs=pl.BlockSpec((tile_b, dim),
                               lambda i: (jnp.where(i < na, 0, i - na), 0)),
        out_shape=jax.ShapeDtypeStruct((t, dim), jnp.float32),
        scratch_shapes=[
            pltpu.VMEM((vp, dim), jnp.float32),
            pltpu.VMEM((vp, dim), jnp.bfloat16),
            pltpu.VMEM((vp, t), jnp.float32),
            pltpu.VMEM((vp, t), jnp.bfloat16),
            pltpu.VMEM((vp, t), jnp.float32),
            pltpu.VMEM((t, dim), jnp.float32),
        ],
    )(x.reshape(t, dim), vision.reshape(v, cv),
      attention_mask.reshape(v, t), Wu, Wk, Wv, bu.reshape(1, dim))
    return out.reshape(b, t, dim)


# speedup vs baseline: 1.0466x; 1.0466x over previous
import functools

import jax
import jax.numpy as jnp
from jax.experimental import pallas as pl
from jax.experimental.pallas import tpu as pltpu

_VALID_BIAS = -100.0
_DN_NT = (((1,), (1,)), ((), ()))
_DN_TN = (((0,), (0,)), ((), ()))


def _body(x_ref, vis_ref, mask_ref, wu_ref, wk_ref, wv_ref, bu_ref,
          out_ref, k_ref, v_ref, s_ref, p_ref, w_ref, xs_ref,
          *, nvis, tile_a, na, tile_b):
    i = pl.program_id(0)
    vp, t = s_ref.shape
    dim = x_ref.shape[1]
    scale = jax.lax.rsqrt(jnp.float32(dim))

    @pl.when(i == 0)
    def _setup():
        vis = jnp.concatenate(
            [vis_ref[...], jnp.zeros((vp - nvis, vis_ref.shape[1]),
                                     jnp.float32)], axis=0)
        up = jax.lax.dot_general(vis, wu_ref[...], _DN_NT,
                                 preferred_element_type=jnp.float32)
        up = up + bu_ref[...]
        k_ref[...] = jax.lax.dot_general(up, wk_ref[...], _DN_NT,
                                         preferred_element_type=jnp.float32)
        v_ref[...] = jax.lax.dot_general(up, wv_ref[...], _DN_NT,
                                         preferred_element_type=jnp.float32
                                         ).astype(jnp.bfloat16)
        mask = jnp.concatenate(
            [mask_ref[...].astype(jnp.float32),
             jnp.zeros((vp - nvis, t), jnp.float32)], axis=0)
        cnt = jnp.sum(mask, axis=0, keepdims=True)
        npad = jnp.max(cnt) - cnt
        rowid = jax.lax.broadcasted_iota(jnp.int32, (vp, t), 0)
        w_ref[...] = jnp.where(rowid == nvis, npad, mask)

    @pl.when(i < na)
    def _scores():
        xt = x_ref[...]
        xs_ref[pl.ds(i * tile_a, tile_a), :] = xt
        s_ref[:, pl.ds(i * tile_a, tile_a)] = jax.lax.dot_general(
            k_ref[...], xt, _DN_NT, preferred_element_type=jnp.float32)

    @pl.when(i == na)
    def _normalize():
        w = w_ref[...]
        rowid = jax.lax.broadcasted_iota(jnp.int32, (vp, t), 0)
        bias = jnp.where(rowid < nvis, jnp.float32(_VALID_BIAS),
                         jnp.float32(0.0))
        lm = jnp.where(w > 0, s_ref[...] * scale + bias, -jnp.inf)
        e = w * jnp.exp(lm - jnp.max(lm))
        p_ref[...] = (e / jnp.sum(e)).astype(jnp.bfloat16)

    @pl.when(i >= na)
    def _combine():
        j = i - na
        p = p_ref[:, pl.ds(j * tile_b, tile_b)]
        attn = jax.lax.dot_general(p, v_ref[...], _DN_TN,
                                   preferred_element_type=jnp.float32)
        out_ref[...] = attn + xs_ref[pl.ds(j * tile_b, tile_b), :]


def kernel(x, vision, attention_mask, Wk, Wv, Wu, bu):
    b, t, dim = x.shape
    v = vision.shape[1]
    cv = vision.shape[2]
    vp = ((v + 1 + 7) // 8) * 8
    tile_a = 1024
    tile_b = 1024
    na = t // tile_a
    nb = t // tile_b

    full = lambda shape: pl.BlockSpec(shape, lambda i: (0, 0))
    out = pl.pallas_call(
        functools.partial(_body, nvis=v, tile_a=tile_a, na=na, tile_b=tile_b),
        grid=(na + nb,),
        in_specs=[
            pl.BlockSpec((tile_a, dim),
                         lambda i: (jnp.where(i < na, i, na - 1), 0)),
            full((v, cv)),
            full((v, t)),
            full((dim, cv)),
            full((dim, dim)),
            full((dim, dim)),
            full((1, dim)),
        ],
        out_specs=pl.BlockSpec((tile_b, dim),
                               lambda i: (jnp.where(i < na, 0, i - na), 0)),
        out_shape=jax.ShapeDtypeStruct((t, dim), jnp.float32),
        scratch_shapes=[
            pltpu.VMEM((vp, dim), jnp.float32),
            pltpu.VMEM((vp, dim), jnp.bfloat16),
            pltpu.VMEM((vp, t), jnp.float32),
            pltpu.VMEM((vp, t), jnp.bfloat16),
            pltpu.VMEM((vp, t), jnp.float32),
            pltpu.VMEM((t, dim), jnp.float32),
        ],
    )(x.reshape(t, dim), vision.reshape(v, cv),
      attention_mask.reshape(v, t), Wu, Wk, Wv, bu.reshape(1, dim))
    return out.reshape(b, t, dim)
